# Initial kernel scaffold; baseline (speedup 1.0000x reference)
import jax
import jax.numpy as jnp
from jax.experimental import pallas as pl


def kernel(seq, adj, W0, W1, W2, Wskip, bias, prelu_a):
    raise NotImplementedError("write your pallas kernel here")


# speedup vs baseline: 1.4860x; 1.4860x over previous
import jax
import jax.numpy as jnp
from jax.experimental import pallas as pl


def _prelu(x, a):
    return jnp.where(x >= 0, x, a * x)


def _gcn_kernel(seq_ref, adj_ref, w0_ref, w1_ref, w2_ref, wskip_ref,
                bias_ref, a_ref, out_ref):
    s = seq_ref[0]
    adj = adj_ref[0]
    a = a_ref[0, 0]

    f32 = jnp.float32
    skip = jnp.dot(s, wskip_ref[...].T, preferred_element_type=f32)

    fts = jnp.dot(s, w0_ref[...].T, preferred_element_type=f32)
    out0 = jnp.dot(adj, fts.astype(jnp.bfloat16), preferred_element_type=f32)
    out0 = _prelu(out0 + bias_ref[0, :], a)

    fts = jnp.dot(out0 + skip, w1_ref[...].T, preferred_element_type=f32)
    out1 = jnp.dot(adj, fts.astype(jnp.bfloat16), preferred_element_type=f32)
    out1 = _prelu(out1 + bias_ref[1, :], a)

    fts = jnp.dot(out1 + out0 + skip, w2_ref[...].T, preferred_element_type=f32)
    out2 = jnp.dot(adj, fts.astype(jnp.bfloat16), preferred_element_type=f32)
    out_ref[0] = _prelu(out2 + bias_ref[2, :], a)


def kernel(seq, adj, W0, W1, W2, Wskip, bias, prelu_a):
    B, N, d_in = seq.shape
    d_out = W0.shape[0]
    adj_bf16 = adj.astype(jnp.bfloat16)
    a2d = jnp.reshape(prelu_a, (1, 1))

    grid = (B,)
    full2d = lambda shape: pl.BlockSpec(shape, lambda b: (0, 0))
    return pl.pallas_call(
        _gcn_kernel,
        grid=grid,
        in_specs=[
            pl.BlockSpec((1, N, d_in), lambda b: (b, 0, 0)),
            pl.BlockSpec((1, N, N), lambda b: (b, 0, 0)),
            full2d((d_out, d_in)),
            full2d((d_out, d_out)),
            full2d((d_out, d_out)),
            full2d((d_out, d_in)),
            full2d((3, d_out)),
            full2d((1, 1)),
        ],
        out_specs=pl.BlockSpec((1, N, d_out), lambda b: (b, 0, 0)),
        out_shape=jax.ShapeDtypeStruct((B, N, d_out), jnp.float32),
    )(seq, adj_bf16, W0, W1, W2, Wskip, bias, a2d)


# speedup vs baseline: 1.5008x; 1.0099x over previous
import jax
import jax.numpy as jnp
from jax.experimental import pallas as pl


def _prelu(x, a):
    return jnp.where(x >= 0, x, a * x)


def _gcn_kernel(seq_ref, adj_ref, w0_ref, w1_ref, w2_ref, wskip_ref,
                bias_ref, a_ref, out_ref):
    s = seq_ref[0]
    adj = adj_ref[0]
    a = a_ref[0, 0]

    f32 = jnp.float32
    bf16 = jnp.bfloat16

    def mm(x, y):
        return jnp.dot(x.astype(bf16), y.astype(bf16),
                       preferred_element_type=f32)

    skip = mm(s, wskip_ref[...].T)

    fts = mm(s, w0_ref[...].T)
    out0 = mm(adj, fts)
    out0 = _prelu(out0 + bias_ref[0, :], a)

    fts = mm(out0 + skip, w1_ref[...].T)
    out1 = mm(adj, fts)
    out1 = _prelu(out1 + bias_ref[1, :], a)

    fts = mm(out1 + out0 + skip, w2_ref[...].T)
    out2 = mm(adj, fts)
    out_ref[0] = _prelu(out2 + bias_ref[2, :], a)


def kernel(seq, adj, W0, W1, W2, Wskip, bias, prelu_a):
    B, N, d_in = seq.shape
    d_out = W0.shape[0]
    adj_bf16 = adj.astype(jnp.bfloat16)
    a2d = jnp.reshape(prelu_a, (1, 1))

    grid = (B,)
    full2d = lambda shape: pl.BlockSpec(shape, lambda b: (0, 0))
    return pl.pallas_call(
        _gcn_kernel,
        grid=grid,
        in_specs=[
            pl.BlockSpec((1, N, d_in), lambda b: (b, 0, 0)),
            pl.BlockSpec((1, N, N), lambda b: (b, 0, 0)),
            full2d((d_out, d_in)),
            full2d((d_out, d_out)),
            full2d((d_out, d_out)),
            full2d((d_out, d_in)),
            full2d((3, d_out)),
            full2d((1, 1)),
        ],
        out_specs=pl.BlockSpec((1, N, d_out), lambda b: (b, 0, 0)),
        out_shape=jax.ShapeDtypeStruct((B, N, d_out), jnp.float32),
    )(seq, adj_bf16, W0, W1, W2, Wskip, bias, a2d)


# speedup vs baseline: 2.1957x; 1.4631x over previous
import jax
import jax.numpy as jnp
from jax.experimental import pallas as pl


def _prelu(x, a):
    return jnp.where(x >= 0, x, a * x)


def _gcn_kernel(seq_ref, adj_ref, w0_ref, w1_ref, w2_ref, wskip_ref,
                bias_ref, a_ref, out_ref):
    a = a_ref[0, 0]
    f32 = jnp.float32
    bf16 = jnp.bfloat16

    adj = adj_ref[0].astype(bf16)
    s = seq_ref[0].astype(bf16)

    def mm(x, y):
        return jnp.dot(x, y, preferred_element_type=f32)

    skip = mm(s, wskip_ref[...].T.astype(bf16))

    fts = mm(s, w0_ref[...].T.astype(bf16)).astype(bf16)
    out0 = mm(adj, fts)
    out0 = _prelu(out0 + bias_ref[0, :], a)

    t = (out0 + skip).astype(bf16)
    fts = mm(t, w1_ref[...].T.astype(bf16)).astype(bf16)
    out1 = mm(adj, fts)
    out1 = _prelu(out1 + bias_ref[1, :], a).astype(bf16)

    fts = mm((out1 + t).astype(bf16), w2_ref[...].T.astype(bf16)).astype(bf16)
    out2 = mm(adj, fts)
    out_ref[0] = _prelu(out2 + bias_ref[2, :], a)


def kernel(seq, adj, W0, W1, W2, Wskip, bias, prelu_a):
    B, N, d_in = seq.shape
    d_out = W0.shape[0]
    a2d = jnp.reshape(prelu_a, (1, 1))

    full2d = lambda shape: pl.BlockSpec(shape, lambda b: (0, 0))
    return pl.pallas_call(
        _gcn_kernel,
        grid=(B,),
        in_specs=[
            pl.BlockSpec((1, N, d_in), lambda b: (b, 0, 0)),
            pl.BlockSpec((1, N, N), lambda b: (b, 0, 0)),
            full2d((d_out, d_in)),
            full2d((d_out, d_out)),
            full2d((d_out, d_out)),
            full2d((d_out, d_in)),
            full2d((3, d_out)),
            full2d((1, 1)),
        ],
        out_specs=pl.BlockSpec((1, N, d_out), lambda b: (b, 0, 0)),
        out_shape=jax.ShapeDtypeStruct((B, N, d_out), jnp.float32),
    )(seq, adj, W0, W1, W2, Wskip, bias, a2d)
